# baseline (speedup 1.0000x reference)
import functools

import jax
import jax.numpy as jnp
from jax import lax
from jax.experimental import pallas as pl
from jax.experimental.pallas import tpu as pltpu
from jax.experimental.pallas import tpu_sc as plsc

_TILING = pltpu.CompilerParams(use_tc_tiling_on_sc=True, needs_layout_passes=False)


def _iota16():
    return lax.iota(jnp.int32, 16)


@functools.cache
def _build_transpose(V, D):
    info = plsc.get_sparse_core_info()
    NC, NS = info.num_cores, info.num_subcores
    NW = NC * NS
    nfull = (V // 128) * 128
    nblk = nfull // 128
    per_w = nblk // NW
    extra = nblk - per_w * NW
    ntail = V - nfull
    assert D == 64 and V % 2 == 0 and ntail % 2 == 0

    mesh = plsc.VectorSubcoreMesh(core_axis_name="c", subcore_axis_name="s")

    @functools.partial(
        pl.kernel,
        mesh=mesh,
        out_type=jax.ShapeDtypeStruct((V // 2, 128), jnp.float32),
        scratch_types=[pltpu.VMEM((D, 128), jnp.float32)] * 2
        + [pltpu.VMEM((64, 128), jnp.float32)] * 2
        + [pltpu.VMEM((ntail, D), jnp.float32)]
        + [pltpu.SemaphoreType.DMA] * 4,
        compiler_params=_TILING,
    )
    def tra(wt_hbm, wtail_hbm, wp_hbm, tb0, tb1, ob0, ob1, tailv, g0, g1, s0, s1):
        tbs, obs, gs, ss = (tb0, tb1), (ob0, ob1), (g0, g1), (s0, s1)
        wid = lax.axis_index("s") * NC + lax.axis_index("c")
        base = wid * per_w
        rowg = [_iota16() + (g * 16) for g in range(D // 16)]

        def fetch(b, p):
            pltpu.async_copy(wt_hbm.at[:, pl.ds(b * 128, 128)], tbs[p], gs[p])

        def flush(b, p):
            pltpu.async_copy(obs[p], wp_hbm.at[pl.ds(b * 64, 64), :], ss[p])

        def transpose_block(b, p):
            pltpu.make_async_copy(wt_hbm.at[:, pl.ds(b * 128, 128)], tbs[p], gs[p]).wait()

            def rbody(r, carry):
                for h in range(2):
                    col = jnp.full((16,), 2 * r + h, jnp.int32)
                    for g in range(D // 16):
                        vals = plsc.load_gather(tbs[p], [rowg[g], col])
                        obs[p][r, pl.ds(h * 64 + g * 16, 16)] = vals
                return carry

            lax.fori_loop(0, 64, rbody, 0)

        if per_w > 0:
            fetch(base, 0)
            if per_w > 1:
                fetch(base + 1, 1)

            def outer(k, carry):
                for p in range(2):
                    b = base + 2 * k + p

                    @pl.when(b < base + per_w)
                    def _():
                        @pl.when(2 * k + p >= 2)
                        def _():
                            pltpu.make_async_copy(
                                obs[p], wp_hbm.at[pl.ds(0, 64), :], ss[p]
                            ).wait()

                        transpose_block(b, p)
                        flush(b, p)

                        @pl.when(b + 2 < base + per_w)
                        def _():
                            fetch(b + 2, p)

                return carry

            lax.fori_loop(0, (per_w + 1) // 2, outer, 0)
            for p in range(min(per_w, 2)):
                pltpu.make_async_copy(
                    obs[p], wp_hbm.at[pl.ds(0, 64), :], ss[p]
                ).wait()

        for e in range(extra):

            @pl.when(wid == e)
            def _():
                b = NW * per_w + e
                fetch(b, 0)
                transpose_block(b, 0)
                flush(b, 0)
                pltpu.make_async_copy(obs[0], wp_hbm.at[pl.ds(0, 64), :], ss[0]).wait()

        if ntail:

            @pl.when(wid == NW - 1)
            def _():
                pltpu.sync_copy(wtail_hbm, tailv)

                def tbody(r, carry):
                    for h in range(2):
                        row = jnp.full((16,), 2 * r + h, jnp.int32)
                        for g in range(D // 16):
                            vals = plsc.load_gather(tailv, [row, rowg[g]])
                            ob0[r, pl.ds(h * 64 + g * 16, 16)] = vals
                    return carry

                lax.fori_loop(0, ntail // 2, tbody, 0)
                pltpu.sync_copy(
                    ob0.at[pl.ds(0, ntail // 2), :],
                    wp_hbm.at[pl.ds(nfull // 2, ntail // 2), :],
                )

    return tra


@functools.cache
def _build_gather(V, D, NI, NJ, scale):
    info = plsc.get_sparse_core_info()
    NC, NS = info.num_cores, info.num_subcores
    NW = NC * NS
    assert NI % (128 * NW) == 0 or NI == 128 * NW
    assert D == 64

    mesh = plsc.VectorSubcoreMesh(core_axis_name="c", subcore_axis_name="s")

    @functools.partial(
        pl.kernel,
        mesh=mesh,
        out_type=jax.ShapeDtypeStruct((NJ, D, NI), jnp.float32),
        scratch_types=[pltpu.VMEM((NJ, 128), jnp.int32)]
        + [pltpu.VMEM((128,), jnp.int32)] * 2
        + [pltpu.VMEM((16,), jnp.int32)] * 2
        + [pltpu.VMEM((128, 128), jnp.float32)] * 2
        + [pltpu.VMEM((1, D, 128), jnp.float32)] * 2
        + [pltpu.SemaphoreType.DMA] * 4,
        compiler_params=_TILING,
    )
    def gat(xt_hbm, wp_hbm, o_hbm, xv, q0, q1, cb0, cb1, r0, r1, sl0, sl1, g0, g1, s0, s1):
        qs, cbs, rbs, sls = (q0, q1), (cb0, cb1), (r0, r1), (sl0, sl1)
        gs, ss = (g0, g1), (s0, s1)
        wid = lax.axis_index("s") * NC + lax.axis_index("c")
        i0 = wid * 128
        rowg = [_iota16() + (g * 16) for g in range(8)]

        pltpu.sync_copy(xt_hbm.at[:, pl.ds(i0, 128)], xv)

        def prep_and_fire(j, p):
            for g in range(8):
                qs[p][pl.ds(g * 16, 16)] = xv[j, pl.ds(g * 16, 16)]
            pltpu.async_copy(wp_hbm.at[qs[p]], rbs[p], gs[p])

        def consume(j, p):
            pltpu.make_async_copy(wp_hbm.at[qs[p]], rbs[p], gs[p]).wait()
            zero = cbs[p][pl.ds(0, 16)] * 0
            for g in range(8):
                for d in range(D):
                    vals = plsc.load_gather(rbs[p], [rowg[g], zero + d])
                    sls[p][0, d, pl.ds(g * 16, 16)] = vals * scale
            pltpu.async_copy(sls[p], o_hbm.at[pl.ds(j, 1), :, pl.ds(i0, 128)], ss[p])

        prep_and_fire(0, 0)
        prep_and_fire(1, 1)

        def outer_guarded(k, carry):
            for p in range(2):
                j = 2 * k + p
                @pl.when(j >= 2)
                def _():
                    pltpu.make_async_copy(
                        sls[p], o_hbm.at[pl.ds(0, 1), :, pl.ds(i0, 128)], ss[p]
                    ).wait()

                consume(j, p)

                @pl.when(j + 2 < NJ)
                def _():
                    prep_and_fire(j + 2, p)

            return carry

        lax.fori_loop(0, NJ // 2, outer_guarded, 0)
        for p in range(2):
            pltpu.make_async_copy(
                sls[p], o_hbm.at[pl.ds(0, 1), :, pl.ds(i0, 128)], ss[p]
            ).wait()

    return gat


def kernel(X, weight):
    NI, NJ = X.shape
    V, D = weight.shape
    scale = float(D) ** 0.5
    nfull = (V // 128) * 128

    WP = jnp.pad(weight, ((0, 0), (0, D)))
    XT = X.T
    O = _build_gather(V, D, NI, NJ, scale)(XT, WP)
    return jnp.transpose(O, (2, 0, 1))


# speedup vs baseline: 1.6494x; 1.6494x over previous
import functools

import jax
import jax.numpy as jnp
from jax import lax
from jax.experimental import pallas as pl
from jax.experimental.pallas import tpu as pltpu
from jax.experimental.pallas import tpu_sc as plsc

_TILING = pltpu.CompilerParams(use_tc_tiling_on_sc=True, needs_layout_passes=False)


def _iota16():
    return lax.iota(jnp.int32, 16)


@functools.cache
def _build_transpose(V, D):
    info = plsc.get_sparse_core_info()
    NC, NS = info.num_cores, info.num_subcores
    NW = NC * NS
    nfull = (V // 128) * 128
    nblk = nfull // 128
    per_w = nblk // NW
    extra = nblk - per_w * NW
    ntail = V - nfull
    assert D == 64 and V % 2 == 0 and ntail % 2 == 0

    mesh = plsc.VectorSubcoreMesh(core_axis_name="c", subcore_axis_name="s")

    @functools.partial(
        pl.kernel,
        mesh=mesh,
        out_type=jax.ShapeDtypeStruct((V // 2, 128), jnp.float32),
        scratch_types=[pltpu.VMEM((D, 128), jnp.float32)] * 2
        + [pltpu.VMEM((64, 128), jnp.float32)] * 2
        + [pltpu.VMEM((ntail, D), jnp.float32)]
        + [pltpu.SemaphoreType.DMA] * 4,
        compiler_params=_TILING,
    )
    def tra(wt_hbm, wtail_hbm, wp_hbm, tb0, tb1, ob0, ob1, tailv, g0, g1, s0, s1):
        tbs, obs, gs, ss = (tb0, tb1), (ob0, ob1), (g0, g1), (s0, s1)
        wid = lax.axis_index("s") * NC + lax.axis_index("c")
        base = wid * per_w
        rowg = [_iota16() + (g * 16) for g in range(D // 16)]

        def fetch(b, p):
            pltpu.async_copy(wt_hbm.at[:, pl.ds(b * 128, 128)], tbs[p], gs[p])

        def flush(b, p):
            pltpu.async_copy(obs[p], wp_hbm.at[pl.ds(b * 64, 64), :], ss[p])

        def transpose_block(b, p):
            pltpu.make_async_copy(wt_hbm.at[:, pl.ds(b * 128, 128)], tbs[p], gs[p]).wait()

            def rbody(r, carry):
                for h in range(2):
                    col = jnp.full((16,), 2 * r + h, jnp.int32)
                    for g in range(D // 16):
                        vals = plsc.load_gather(tbs[p], [rowg[g], col])
                        obs[p][r, pl.ds(h * 64 + g * 16, 16)] = vals
                return carry

            lax.fori_loop(0, 64, rbody, 0)

        if per_w > 0:
            fetch(base, 0)
            if per_w > 1:
                fetch(base + 1, 1)

            def outer(k, carry):
                for p in range(2):
                    b = base + 2 * k + p

                    @pl.when(b < base + per_w)
                    def _():
                        @pl.when(2 * k + p >= 2)
                        def _():
                            pltpu.make_async_copy(
                                obs[p], wp_hbm.at[pl.ds(0, 64), :], ss[p]
                            ).wait()

                        transpose_block(b, p)
                        flush(b, p)

                        @pl.when(b + 2 < base + per_w)
                        def _():
                            fetch(b + 2, p)

                return carry

            lax.fori_loop(0, (per_w + 1) // 2, outer, 0)
            for p in range(min(per_w, 2)):
                pltpu.make_async_copy(
                    obs[p], wp_hbm.at[pl.ds(0, 64), :], ss[p]
                ).wait()

        for e in range(extra):

            @pl.when(wid == e)
            def _():
                b = NW * per_w + e
                fetch(b, 0)
                transpose_block(b, 0)
                flush(b, 0)
                pltpu.make_async_copy(obs[0], wp_hbm.at[pl.ds(0, 64), :], ss[0]).wait()

        if ntail:

            @pl.when(wid == NW - 1)
            def _():
                pltpu.sync_copy(wtail_hbm, tailv)

                def tbody(r, carry):
                    for h in range(2):
                        row = jnp.full((16,), 2 * r + h, jnp.int32)
                        for g in range(D // 16):
                            vals = plsc.load_gather(tailv, [row, rowg[g]])
                            ob0[r, pl.ds(h * 64 + g * 16, 16)] = vals
                    return carry

                lax.fori_loop(0, ntail // 2, tbody, 0)
                pltpu.sync_copy(
                    ob0.at[pl.ds(0, ntail // 2), :],
                    wp_hbm.at[pl.ds(nfull // 2, ntail // 2), :],
                )

    return tra


@functools.cache
def _build_gather(V, D, NI, NJ, scale):
    info = plsc.get_sparse_core_info()
    NC, NS = info.num_cores, info.num_subcores
    NW = NC * NS
    assert NI % (128 * NW) == 0 or NI == 128 * NW
    assert D == 64

    mesh = plsc.VectorSubcoreMesh(core_axis_name="c", subcore_axis_name="s")

    @functools.partial(
        pl.kernel,
        mesh=mesh,
        out_type=jax.ShapeDtypeStruct((NJ, D, NI), jnp.float32),
        scratch_types=[pltpu.VMEM((NJ, 128), jnp.int32)]
        + [pltpu.VMEM((128,), jnp.int32)] * 2
        + [pltpu.VMEM((128, 128), jnp.float32)] * 2
        + [pltpu.VMEM((D, 256), jnp.float32)]
        + [pltpu.VMEM((1, D, 128), jnp.float32)] * 2
        + [pltpu.SemaphoreType.DMA] * 4,
        compiler_params=_TILING,
    )
    def gat(xt_hbm, wp_hbm, o_hbm, xv, q0, q1, r0, r1, slabx, sl0, sl1, g0, g1, s0, s1):
        qs, rbs, sls = (q0, q1), (r0, r1), (sl0, sl1)
        gs, ss = (g0, g1), (s0, s1)
        wid = lax.axis_index("s") * NC + lax.axis_index("c")
        i0 = wid * 128
        dvec = [_iota16() + (k * 16) for k in range(4)]

        pltpu.sync_copy(xt_hbm.at[:, pl.ds(i0, 128)], xv)

        def prep_and_fire(j, p):
            for g in range(8):
                qs[p][pl.ds(g * 16, 16)] = xv[j, pl.ds(g * 16, 16)]
            pltpu.async_copy(wp_hbm.at[qs[p]], rbs[p], gs[p])

        def consume(j, p):
            pltpu.make_async_copy(wp_hbm.at[qs[p]], rbs[p], gs[p]).wait()

            def p1(il, carry):
                for k in range(4):
                    vals = rbs[p][il, pl.ds(k * 16, 16)] * scale
                    colv = (dvec[k] + il) & 127
                    plsc.store_scatter(slabx, [dvec[k], colv], vals)
                    plsc.store_scatter(slabx, [dvec[k], colv + 128], vals)
                return carry

            lax.fori_loop(0, 128, p1, 0)

            def p2(d, carry):
                for k in range(4):
                    sls[p][0, d, pl.ds(k * 16, 16)] = slabx[d, pl.ds(d + k * 16, 16)]
                return carry

            lax.fori_loop(0, D, p2, 0)
            pltpu.async_copy(sls[p], o_hbm.at[pl.ds(j, 1), :, pl.ds(i0, 128)], ss[p])

        prep_and_fire(0, 0)
        prep_and_fire(1, 1)

        def outer_guarded(k, carry):
            for p in range(2):
                j = 2 * k + p
                @pl.when(j >= 2)
                def _():
                    pltpu.make_async_copy(
                        sls[p], o_hbm.at[pl.ds(0, 1), :, pl.ds(i0, 128)], ss[p]
                    ).wait()

                consume(j, p)

                @pl.when(j + 2 < NJ)
                def _():
                    prep_and_fire(j + 2, p)

            return carry

        lax.fori_loop(0, NJ // 2, outer_guarded, 0)
        for p in range(2):
            pltpu.make_async_copy(
                sls[p], o_hbm.at[pl.ds(0, 1), :, pl.ds(i0, 128)], ss[p]
            ).wait()

    return gat


def kernel(X, weight):
    NI, NJ = X.shape
    V, D = weight.shape
    scale = float(D) ** 0.5
    nfull = (V // 128) * 128

    WP = jnp.pad(weight, ((0, 0), (0, D)))
    XT = X.T
    O = _build_gather(V, D, NI, NJ, scale)(XT, WP)
    return jnp.transpose(O, (2, 0, 1))
